# baseline (speedup 1.0000x reference)
import functools

import jax
import jax.numpy as jnp
from jax import lax
from jax.experimental import pallas as pl
from jax.experimental.pallas import tpu as pltpu
from jax.experimental.pallas import tpu_sc as plsc

_HID = 4096
_NE = 64
_K = 8
_BT = 1024

_NCORE = 2
_NSUB = 16
_NW = _NCORE * _NSUB
_CH = 64


def _matmul_block(x_ref, wt_ref, l_ref):
    l_ref[...] = jnp.dot(x_ref[...], wt_ref[...],
                         preferred_element_type=jnp.float32)


def _tc_logits(flat, wt, block0, n_blocks):
    return pl.pallas_call(
        _matmul_block,
        grid=(n_blocks,),
        in_specs=[
            pl.BlockSpec((_BT, _HID), lambda i: (block0 + i, 0)),
            pl.BlockSpec((_HID, _NE), lambda i: (0, 0)),
        ],
        out_specs=pl.BlockSpec((_BT, _NE), lambda i: (i, 0)),
        out_shape=jax.ShapeDtypeStruct((n_blocks * _BT, _NE), jnp.float32),
    )(flat, wt)


def _sc_body(n_tok, logits_hbm, rw_hbm, se_hbm, chunk_v, outw_v, outi_v):
    tpw = n_tok // _NW
    nch = tpw // _CH
    wid = lax.axis_index("s") * _NCORE + lax.axis_index("c")
    base = wid * tpw
    lane = lax.iota(jnp.int32, 16)
    low = lane < _K

    def merge(ka, ia, kb, ib):
        krb = lax.rev(kb, (0,))
        irb = lax.rev(ib, (0,))
        gt = ka >= krb
        kk = jnp.where(gt, ka, krb)
        ii = jnp.where(gt, ia, irb)
        return plsc.sort_key_val(kk, ii, descending=True)

    for ch in range(nch):
        tok0 = base + ch * _CH
        pltpu.sync_copy(logits_hbm.at[pl.ds(tok0 * _NE, _CH * _NE)], chunk_v)

        def body(t, carry):
            off = t * _NE
            ks = []
            vs = []
            for j in range(4):
                kj = chunk_v[pl.ds(off + 16 * j, 16)]
                ij = lane + 16 * j
                kjs, ijs = plsc.sort_key_val(kj, ij, descending=True)
                ks.append(kjs)
                vs.append(ijs)
            k01, i01 = merge(ks[0], vs[0], ks[1], vs[1])
            k23, i23 = merge(ks[2], vs[2], ks[3], vs[3])
            kf, idf = merge(k01, i01, k23, i23)
            m = jnp.max(kf)
            e = jnp.exp(kf - m)
            e = jnp.where(low, e, 0.0)
            w = e / jnp.sum(e)
            plsc.store_compressed(outw_v.at[pl.ds(t * _K, 16)], w, mask=low)
            plsc.store_compressed(outi_v.at[pl.ds(t * _K, 16)], idf, mask=low)
            return carry

        lax.fori_loop(0, _CH, body, 0)
        pltpu.sync_copy(outw_v.at[pl.ds(0, _CH * _K)],
                        rw_hbm.at[pl.ds(tok0 * _K, _CH * _K)])
        pltpu.sync_copy(outi_v.at[pl.ds(0, _CH * _K)],
                        se_hbm.at[pl.ds(tok0 * _K, _CH * _K)])


def _sc_topk(logits):
    n_tok = logits.shape[0]
    mesh = plsc.VectorSubcoreMesh(core_axis_name="c", subcore_axis_name="s")
    fn = pl.kernel(
        functools.partial(_sc_body, n_tok),
        out_type=[
            jax.ShapeDtypeStruct((n_tok * _K,), jnp.float32),
            jax.ShapeDtypeStruct((n_tok * _K,), jnp.int32),
        ],
        mesh=mesh,
        scratch_types=[
            pltpu.VMEM((_CH * _NE,), jnp.float32),
            pltpu.VMEM((_CH * _K + _K,), jnp.float32),
            pltpu.VMEM((_CH * _K + _K,), jnp.int32),
        ],
        compiler_params=pltpu.CompilerParams(needs_layout_passes=False),
    )
    rw, se = fn(logits.reshape(n_tok * _NE))
    return rw.reshape(n_tok, _K), se.reshape(n_tok, _K)


def kernel(hidden_states, gate_w):
    flat = hidden_states.reshape(-1, _HID)
    n_tok = flat.shape[0]
    wt = gate_w.T
    n_chunks = 4
    ct = n_tok // n_chunks
    rws = []
    ses = []
    for c in range(n_chunks):
        logits = _tc_logits(flat, wt, c * (ct // _BT), ct // _BT)
        rw, se = _sc_topk(logits)
        rws.append(rw)
        ses.append(se)
    return (jnp.concatenate(rws, axis=0), jnp.concatenate(ses, axis=0))


# speedup vs baseline: 1.5007x; 1.5007x over previous
import jax
import jax.numpy as jnp
from jax.experimental import pallas as pl

_HID = 4096
_NE = 64
_K = 8
_BT = 1024


def _router_block(x_ref, wt_ref, rw_ref, se_ref):
    x = x_ref[...]
    wt = wt_ref[...]
    logits = jax.lax.dot_general(x, wt, (((1,), (1,)), ((), ())), preferred_element_type=jnp.float32)
    cur = logits.T
    row = jax.lax.broadcasted_iota(jnp.int32, cur.shape, 0)
    vals = []
    idxs = []
    for _ in range(_K):
        m = jnp.max(cur, axis=0, keepdims=True)
        idx = jnp.min(jnp.where(cur == m, row, _NE), axis=0, keepdims=True)
        vals.append(m)
        idxs.append(idx)
        cur = jnp.where(row == idx, -jnp.inf, cur)
    v = jnp.concatenate(vals, axis=0)
    i = jnp.concatenate(idxs, axis=0)
    e = jnp.exp(v - v[:1])
    w = e / jnp.sum(e, axis=0, keepdims=True)
    rw_ref[...] = w.T
    se_ref[...] = i.T


def kernel(hidden_states, gate_w):
    flat = hidden_states.reshape(-1, _HID)
    n_tok = flat.shape[0]
    wt = gate_w
    rw, se = pl.pallas_call(
        _router_block,
        grid=(n_tok // _BT,),
        in_specs=[
            pl.BlockSpec((_BT, _HID), lambda i: (i, 0)),
            pl.BlockSpec((_NE, _HID), lambda i: (0, 0)),
        ],
        out_specs=[
            pl.BlockSpec((_BT, _K), lambda i: (i, 0)),
            pl.BlockSpec((_BT, _K), lambda i: (i, 0)),
        ],
        out_shape=[
            jax.ShapeDtypeStruct((n_tok, _K), jnp.float32),
            jax.ShapeDtypeStruct((n_tok, _K), jnp.int32),
        ],
    )(flat, wt)
    return (rw, se)
